# baseline (speedup 1.0000x reference)
import functools

import jax
import jax.numpy as jnp
from jax import lax
from jax.experimental import pallas as pl
from jax.experimental.pallas import tpu as pltpu
from jax.experimental.pallas import tpu_sc as plsc

D_MODEL = 64
LANES = 16
CHUNK = 128
SCALE = 8.0
NBUF = 4
NW = 32


def _make_sc_gather(n_seq: int, n_btile: int):
    mesh = plsc.VectorSubcoreMesh(core_axis_name="c", subcore_axis_name="s")
    s_tiles = n_seq // 8

    @functools.partial(
        pl.kernel,
        mesh=mesh,
        out_type=jax.ShapeDtypeStruct((n_seq, 8, n_btile, 8, CHUNK),
                                      jnp.float32),
        scratch_types=[
            pltpu.VMEM((s_tiles, 8, CHUNK), jnp.int32),
            [pltpu.VMEM((CHUNK, D_MODEL), jnp.float32)] * NBUF,
            [pltpu.VMEM((8, 8, CHUNK), jnp.float32)] * NBUF,
            [pltpu.SemaphoreType.DMA] * NBUF,
            [pltpu.SemaphoreType.DMA] * NBUF,
        ],
        compiler_params=pltpu.CompilerParams(use_tc_tiling_on_sc=False,
                                             needs_layout_passes=False),
    )
    def k(lut_hbm, idx_hbm, out_hbm, idx_v, inb, outb, gsem, osem):
        w = lax.axis_index("s") * 2 + lax.axis_index("c")
        pltpu.sync_copy(idx_hbm.at[:, w], idx_v)
        iota = lax.iota(jnp.int32, 16)

        for b in range(NBUF):
            pltpu.make_async_copy(
                lut_hbm.at[idx_v.at[b // 8, b % 8]], inb[b], gsem[b]).start()

        def group_body(g, _):
            for b in range(NBUF):
                step = g * NBUF + b

                @pl.when(g > 0)
                def _wait_store():
                    pltpu.make_async_copy(
                        outb[b], out_hbm.at[step - NBUF, :, w],
                        osem[b]).wait()

                pltpu.make_async_copy(
                    lut_hbm.at[idx_v.at[step // 8, step % 8]], inb[b],
                    gsem[b]).wait()

                def d_body(dh, _, b=b):
                    for dl in range(8):
                        dvec = jnp.zeros((LANES,), jnp.int32) + (dh * 8 + dl)
                        for blk in range(CHUNK // LANES):
                            v = plsc.load_gather(
                                inb[b], [iota + blk * LANES, dvec])
                            outb[b][dh, dl, pl.ds(blk * LANES, LANES)] = (
                                v * SCALE)
                    return 0

                lax.fori_loop(0, 8, d_body, 0)

                @pl.when(step + NBUF < n_seq)
                def _next_gather():
                    nxt = step + NBUF
                    pltpu.make_async_copy(
                        lut_hbm.at[idx_v.at[nxt // 8, nxt % 8]], inb[b],
                        gsem[b]).start()

                pltpu.make_async_copy(
                    outb[b], out_hbm.at[step, :, w], osem[b]).start()
            return 0

        lax.fori_loop(0, n_seq // NBUF, group_body, 0)

        for b in range(NBUF):
            pltpu.make_async_copy(
                outb[b], out_hbm.at[n_seq - NBUF + b, :, w], osem[b]).wait()

    return k


def kernel(x, lut):
    b, s = x.shape
    n_btile = b // CHUNK
    x5 = (x.astype(jnp.int32).T
          .reshape(s // 8, 8, n_btile, CHUNK).transpose(0, 2, 1, 3))
    out5 = _make_sc_gather(s, n_btile)(lut, x5)
    return out5.transpose(2, 4, 0, 1, 3).reshape(b, s, D_MODEL)


# speedup vs baseline: 1.5928x; 1.5928x over previous
import functools

import jax
import jax.numpy as jnp
from jax import lax
from jax.experimental import pallas as pl
from jax.experimental.pallas import tpu as pltpu
from jax.experimental.pallas import tpu_sc as plsc

D_MODEL = 64
LANES = 16
CHUNK = 128
SCALE = 8.0
NBUF = 4
NW = 32


def _make_sc_gather(n_seq: int, n_btile: int):
    mesh = plsc.VectorSubcoreMesh(core_axis_name="c", subcore_axis_name="s")
    s_tiles = n_seq // 8

    @functools.partial(
        pl.kernel,
        mesh=mesh,
        out_type=jax.ShapeDtypeStruct((n_seq, 8, n_btile, 8, CHUNK),
                                      jnp.float32),
        scratch_types=[
            pltpu.VMEM((s_tiles, 8, CHUNK), jnp.int32),
            [pltpu.VMEM((CHUNK, D_MODEL), jnp.float32)] * NBUF,
            [pltpu.VMEM((8, 8, CHUNK), jnp.float32)] * NBUF,
            [pltpu.SemaphoreType.DMA] * NBUF,
            [pltpu.SemaphoreType.DMA] * NBUF,
        ],
        compiler_params=pltpu.CompilerParams(use_tc_tiling_on_sc=False,
                                             needs_layout_passes=False),
    )
    def k(lut_hbm, idx_hbm, out_hbm, idx_v, inb, outb, gsem, osem):
        w = lax.axis_index("s") * 2 + lax.axis_index("c")
        pltpu.sync_copy(idx_hbm.at[:, w], idx_v)
        iota = lax.iota(jnp.int32, 16)
        blkvecs = [iota + blk * LANES for blk in range(CHUNK // LANES)]

        for b in range(NBUF):
            pltpu.make_async_copy(
                lut_hbm.at[idx_v.at[b // 8, b % 8]], inb[b], gsem[b]).start()

        def group_body(g, _):
            for b in range(NBUF):
                step = g * NBUF + b

                @pl.when(g > 0)
                def _wait_store():
                    pltpu.make_async_copy(
                        outb[b], out_hbm.at[step - NBUF, :, w],
                        osem[b]).wait()

                pltpu.make_async_copy(
                    lut_hbm.at[idx_v.at[step // 8, step % 8]], inb[b],
                    gsem[b]).wait()

                @plsc.parallel_loop(0, D_MODEL, unroll=2)
                def d_body(d, b=b):
                    dvec = jnp.zeros((LANES,), jnp.int32) + d
                    dh = d // 8
                    dl = d % 8
                    for blk in range(CHUNK // LANES):
                        v = plsc.load_gather(inb[b], [blkvecs[blk], dvec])
                        outb[b][dh, dl, pl.ds(blk * LANES, LANES)] = (
                            v * SCALE)

                @pl.when(step + NBUF < n_seq)
                def _next_gather():
                    nxt = step + NBUF
                    pltpu.make_async_copy(
                        lut_hbm.at[idx_v.at[nxt // 8, nxt % 8]], inb[b],
                        gsem[b]).start()

                pltpu.make_async_copy(
                    outb[b], out_hbm.at[step, :, w], osem[b]).start()
            return 0

        lax.fori_loop(0, n_seq // NBUF, group_body, 0)

        for b in range(NBUF):
            pltpu.make_async_copy(
                outb[b], out_hbm.at[n_seq - NBUF + b, :, w], osem[b]).wait()

    return k


def kernel(x, lut):
    b, s = x.shape
    n_btile = b // CHUNK
    x5 = (x.astype(jnp.int32).T
          .reshape(s // 8, 8, n_btile, CHUNK).transpose(0, 2, 1, 3))
    out5 = _make_sc_gather(s, n_btile)(lut, x5)
    return out5.transpose(2, 4, 0, 1, 3).reshape(b, s, D_MODEL)


# speedup vs baseline: 1.6016x; 1.0055x over previous
import functools

import jax
import jax.numpy as jnp
from jax import lax
from jax.experimental import pallas as pl
from jax.experimental.pallas import tpu as pltpu
from jax.experimental.pallas import tpu_sc as plsc

D_MODEL = 64
LANES = 16
CHUNK = 128
SCALE = 8.0
NBUF = 4
NW = 32


def _make_sc_gather(n_seq: int, n_btile: int):
    mesh = plsc.VectorSubcoreMesh(core_axis_name="c", subcore_axis_name="s")
    s_tiles = n_seq // 8

    @functools.partial(
        pl.kernel,
        mesh=mesh,
        out_type=jax.ShapeDtypeStruct((n_seq, 8, n_btile, 8, CHUNK),
                                      jnp.float32),
        scratch_types=[
            pltpu.VMEM((s_tiles, 8, CHUNK), jnp.int32),
            [pltpu.VMEM((CHUNK, D_MODEL), jnp.float32)] * NBUF,
            [pltpu.VMEM((8, 8, CHUNK), jnp.float32)] * NBUF,
            [pltpu.SemaphoreType.DMA] * NBUF,
            [pltpu.SemaphoreType.DMA] * NBUF,
        ],
        compiler_params=pltpu.CompilerParams(use_tc_tiling_on_sc=False,
                                             needs_layout_passes=False),
    )
    def k(lut_hbm, idx_hbm, out_hbm, idx_v, inb, outb, gsem, osem):
        w = lax.axis_index("s") * 2 + lax.axis_index("c")
        pltpu.sync_copy(idx_hbm.at[:, w], idx_v)
        iota = lax.iota(jnp.int32, 16)
        blkvecs = [iota + blk * LANES for blk in range(CHUNK // LANES)]

        for b in range(NBUF):
            pltpu.make_async_copy(
                lut_hbm.at[idx_v.at[b // 8, b % 8]], inb[b], gsem[b]).start()

        def group_body(g, _):
            for b in range(NBUF):
                step = g * NBUF + b

                @pl.when(g > 0)
                def _wait_store():
                    pltpu.make_async_copy(
                        outb[b], out_hbm.at[step - NBUF, :, w],
                        osem[b]).wait()

                pltpu.make_async_copy(
                    lut_hbm.at[idx_v.at[step // 8, step % 8]], inb[b],
                    gsem[b]).wait()

                @plsc.parallel_loop(0, D_MODEL, unroll=4)
                def d_body(d, b=b):
                    dvec = jnp.zeros((LANES,), jnp.int32) + d
                    dh = d // 8
                    dl = d % 8
                    for blk in range(CHUNK // LANES):
                        v = plsc.load_gather(inb[b], [blkvecs[blk], dvec])
                        outb[b][dh, dl, pl.ds(blk * LANES, LANES)] = (
                            v * SCALE)

                @pl.when(step + NBUF < n_seq)
                def _next_gather():
                    nxt = step + NBUF
                    pltpu.make_async_copy(
                        lut_hbm.at[idx_v.at[nxt // 8, nxt % 8]], inb[b],
                        gsem[b]).start()

                pltpu.make_async_copy(
                    outb[b], out_hbm.at[step, :, w], osem[b]).start()
            return 0

        lax.fori_loop(0, n_seq // NBUF, group_body, 0)

        for b in range(NBUF):
            pltpu.make_async_copy(
                outb[b], out_hbm.at[n_seq - NBUF + b, :, w], osem[b]).wait()

    return k


def kernel(x, lut):
    b, s = x.shape
    n_btile = b // CHUNK
    x5 = (x.astype(jnp.int32).T
          .reshape(s // 8, 8, n_btile, CHUNK).transpose(0, 2, 1, 3))
    out5 = _make_sc_gather(s, n_btile)(lut, x5)
    return out5.transpose(2, 4, 0, 1, 3).reshape(b, s, D_MODEL)


# speedup vs baseline: 2.6383x; 1.6474x over previous
import functools

import jax
import jax.numpy as jnp
from jax import lax
from jax.experimental import pallas as pl
from jax.experimental.pallas import tpu as pltpu
from jax.experimental.pallas import tpu_sc as plsc

D_MODEL = 64
LANES = 16
CHUNK = 128
SCALE = 8.0
NBUF = 4
NW = 32


def _make_sc_gather(n_seq: int, n_btile: int):
    mesh = plsc.VectorSubcoreMesh(core_axis_name="c", subcore_axis_name="s")
    s_tiles = n_seq // 8

    @functools.partial(
        pl.kernel,
        mesh=mesh,
        out_type=jax.ShapeDtypeStruct((n_seq, 8, n_btile, 8, CHUNK),
                                      jnp.float32),
        scratch_types=[
            pltpu.VMEM((s_tiles, 8, CHUNK), jnp.int32),
            pltpu.VMEM((CHUNK, D_MODEL + 1), jnp.float32),
            [pltpu.VMEM((CHUNK, D_MODEL), jnp.float32)] * NBUF,
            [pltpu.VMEM((8, 8, CHUNK), jnp.float32)] * NBUF,
            [pltpu.SemaphoreType.DMA] * NBUF,
            [pltpu.SemaphoreType.DMA] * NBUF,
        ],
        compiler_params=pltpu.CompilerParams(use_tc_tiling_on_sc=False,
                                             needs_layout_passes=False),
    )
    def k(lut_hbm, idx_hbm, out_hbm, idx_v, skb, inb, outb, gsem, osem):
        w = lax.axis_index("s") * 2 + lax.axis_index("c")
        pltpu.sync_copy(idx_hbm.at[:, w], idx_v)
        iota = lax.iota(jnp.int32, 16)
        blkvecs = [iota + blk * LANES for blk in range(CHUNK // LANES)]

        for b in range(NBUF):
            pltpu.make_async_copy(
                lut_hbm.at[idx_v.at[b // 8, b % 8]], inb[b], gsem[b]).start()

        def group_body(g, _):
            for b in range(NBUF):
                step = g * NBUF + b

                @pl.when(g > 0)
                def _wait_store():
                    pltpu.make_async_copy(
                        outb[b], out_hbm.at[step - NBUF, :, w],
                        osem[b]).wait()

                pltpu.make_async_copy(
                    lut_hbm.at[idx_v.at[step // 8, step % 8]], inb[b],
                    gsem[b]).wait()

                @plsc.parallel_loop(0, CHUNK, unroll=2)
                def skew_body(bl, b=b):
                    for kk in range(D_MODEL // LANES):
                        skb[bl, pl.ds(kk * LANES, LANES)] = (
                            inb[b][bl, pl.ds(kk * LANES, LANES)] * SCALE)

                @plsc.parallel_loop(0, D_MODEL, unroll=2)
                def d_body(d, b=b):
                    dvec = jnp.zeros((LANES,), jnp.int32) + d
                    dh = d // 8
                    dl = d % 8
                    for blk in range(CHUNK // LANES):
                        v = plsc.load_gather(skb, [blkvecs[blk], dvec])
                        outb[b][dh, dl, pl.ds(blk * LANES, LANES)] = v

                @pl.when(step + NBUF < n_seq)
                def _next_gather():
                    nxt = step + NBUF
                    pltpu.make_async_copy(
                        lut_hbm.at[idx_v.at[nxt // 8, nxt % 8]], inb[b],
                        gsem[b]).start()

                pltpu.make_async_copy(
                    outb[b], out_hbm.at[step, :, w], osem[b]).start()
            return 0

        lax.fori_loop(0, n_seq // NBUF, group_body, 0)

        for b in range(NBUF):
            pltpu.make_async_copy(
                outb[b], out_hbm.at[n_seq - NBUF + b, :, w], osem[b]).wait()

    return k


def kernel(x, lut):
    b, s = x.shape
    n_btile = b // CHUNK
    x5 = (x.astype(jnp.int32).T
          .reshape(s // 8, 8, n_btile, CHUNK).transpose(0, 2, 1, 3))
    out5 = _make_sc_gather(s, n_btile)(lut, x5)
    return out5.transpose(2, 4, 0, 1, 3).reshape(b, s, D_MODEL)
